# baseline (speedup 1.0000x reference)
import functools

import jax
import jax.numpy as jnp
from jax import lax
from jax.experimental import pallas as pl
from jax.experimental.pallas import tpu as pltpu
from jax.experimental.pallas import tpu_sc as plsc

D = 32
L = 16
NW = 32
IW = 128
G = 5
C = G * IW


def _embed_lookup(idx2d, e1, e2, half):
    n_rows, _ = idx2d.shape
    n = n_rows * IW
    per_w = n // NW
    n_chunks = per_w // C
    assert n_chunks % 2 == 0

    mesh = plsc.VectorSubcoreMesh(core_axis_name="c", subcore_axis_name="s")

    @functools.partial(
        pl.kernel,
        mesh=mesh,
        compiler_params=pltpu.CompilerParams(use_tc_tiling_on_sc=False),
        out_type=jax.ShapeDtypeStruct((n, D), jnp.float32),
        scratch_types=[
            pltpu.VMEM((G, IW), jnp.int32),
            pltpu.VMEM((G, IW), jnp.int32),
            pltpu.VMEM((G, IW), jnp.int32),
            pltpu.VMEM((G, IW), jnp.int32),
            pltpu.VMEM((G, IW), jnp.int32),
            pltpu.VMEM((G, IW), jnp.int32),
            pltpu.VMEM((C, D), jnp.float32),
            pltpu.VMEM((C, D), jnp.float32),
            pltpu.VMEM((C, D), jnp.float32),
            pltpu.VMEM((C, D), jnp.float32),
            pltpu.SemaphoreType.DMA,
            pltpu.SemaphoreType.DMA,
            pltpu.SemaphoreType.DMA,
            pltpu.SemaphoreType.DMA,
            pltpu.SemaphoreType.DMA,
            pltpu.SemaphoreType.DMA,
        ],
    )
    def k(idx_hbm, e1_hbm, e2_hbm, out_hbm,
          x0, x1, a0, a1, b0, b1, p0, p1, q0, q1,
          si0, si1, sg0, sg1, so0, so1):
        wid = lax.axis_index("s") * 2 + lax.axis_index("c")
        base_w = wid * per_w
        rbase_w = base_w // IW

        def issue_idx(ci, xr, si):
            pltpu.async_copy(idx_hbm.at[pl.ds(rbase_w + ci * G, G)], xr, si)

        def wait_idx(xr, si):
            pltpu.make_async_copy(idx_hbm.at[pl.ds(0, G)], xr, si).wait()

        def compute_i(xr, ar, br):
            for j in range(G):
                def grp(g, c):
                    v = xr[j, pl.ds(g * L, L)]
                    ar[j, pl.ds(g * L, L)] = jnp.minimum(v, half - 1)
                    br[j, pl.ds(g * L, L)] = jnp.maximum(v - half, 0)
                    return c
                lax.fori_loop(0, IW // L, grp, 0)

        def issue_gathers(ar, br, pr, qr, sg):
            for j in range(G):
                pltpu.async_copy(e1_hbm.at[ar.at[j]],
                                 pr.at[pl.ds(j * IW, IW)], sg)
                pltpu.async_copy(e2_hbm.at[br.at[j]],
                                 qr.at[pl.ds(j * IW, IW)], sg)

        def wait_gathers(ar, br, pr, qr, sg):
            for j in range(G):
                pltpu.make_async_copy(e1_hbm.at[ar.at[j]],
                                      pr.at[pl.ds(j * IW, IW)], sg).wait()
                pltpu.make_async_copy(e2_hbm.at[br.at[j]],
                                      qr.at[pl.ds(j * IW, IW)], sg).wait()

        def select(xr, pr, qr):
            for j in range(G):
                def sgrp(g, c):
                    iv = xr[j, pl.ds(g * L, L)]
                    rb = j * IW + g * L
                    for t in range(L):
                        r = rb + t
                        take_e1 = iv[t] < half
                        lo = jnp.where(take_e1, pr[r, pl.ds(0, L)],
                                       qr[r, pl.ds(0, L)])
                        hi = jnp.where(take_e1, pr[r, pl.ds(L, L)],
                                       qr[r, pl.ds(L, L)])
                        pr[r, pl.ds(0, L)] = lo
                        pr[r, pl.ds(L, L)] = hi
                    return c
                lax.fori_loop(0, IW // L, sgrp, 0)

        def issue_out(ci, pr, so):
            pltpu.async_copy(pr, out_hbm.at[pl.ds(base_w + ci * C, C)], so)

        def wait_out(pr, so):
            pltpu.make_async_copy(pr, out_hbm.at[pl.ds(0, C)], so).wait()

        sets = (
            (x0, a0, b0, p0, q0, si0, sg0, so0),
            (x1, a1, b1, p1, q1, si1, sg1, so1),
        )

        issue_idx(0, x0, si0)
        wait_idx(x0, si0)
        compute_i(x0, a0, b0)
        issue_gathers(a0, b0, p0, q0, sg0)
        issue_idx(1, x1, si1)

        def iter_body(ci, A, B):
            xA, aA, bA, pA, qA, siA, sgA, soA = A
            xB, aB, bB, pB, qB, siB, sgB, soB = B

            @pl.when(ci + 1 < n_chunks)
            def _():
                wait_idx(xB, siB)
                compute_i(xB, aB, bB)

                @pl.when(ci >= 1)
                def _():
                    wait_out(pB, soB)
                issue_gathers(aB, bB, pB, qB, sgB)

            wait_gathers(aA, bA, pA, qA, sgA)
            select(xA, pA, qA)
            issue_out(ci, pA, soA)

            @pl.when(ci + 2 < n_chunks)
            def _():
                issue_idx(ci + 2, xA, siA)

        def half_body(h, carry):
            iter_body(2 * h, sets[0], sets[1])
            iter_body(2 * h + 1, sets[1], sets[0])
            return carry

        lax.fori_loop(0, n_chunks // 2, half_body, 0)

        wait_out(p0, so0)
        wait_out(p1, so1)

    return k(idx2d, e1, e2)


def kernel(inputs, e1, e2):
    b, h = inputs.shape
    half = e1.shape[0]
    idx2d = inputs.reshape(b * h // IW, IW).astype(jnp.int32)
    out = _embed_lookup(idx2d, e1, e2, half)
    return out.reshape(b, h, D)


# speedup vs baseline: 1.0002x; 1.0002x over previous
import functools

import jax
import jax.numpy as jnp
from jax import lax
from jax.experimental import pallas as pl
from jax.experimental.pallas import tpu as pltpu
from jax.experimental.pallas import tpu_sc as plsc

D = 32
L = 16
NW = 32
IW = 128
G = 5
C = G * IW


def _embed_lookup(idx2d, e1, e2, half):
    n_rows, _ = idx2d.shape
    n = n_rows * IW
    per_w = n // NW
    n_chunks = per_w // C
    assert n_chunks % 2 == 0

    mesh = plsc.VectorSubcoreMesh(core_axis_name="c", subcore_axis_name="s")

    @functools.partial(
        pl.kernel,
        mesh=mesh,
        compiler_params=pltpu.CompilerParams(use_tc_tiling_on_sc=False),
        out_type=jax.ShapeDtypeStruct((n, D), jnp.float32),
        scratch_types=[
            pltpu.VMEM((G, IW), jnp.int32),
            pltpu.VMEM((G, IW), jnp.int32),
            pltpu.VMEM((G, IW), jnp.int32),
            pltpu.VMEM((G, IW), jnp.int32),
            pltpu.VMEM((G, IW), jnp.int32),
            pltpu.VMEM((G, IW), jnp.int32),
            pltpu.VMEM((C, D), jnp.float32),
            pltpu.VMEM((C, D), jnp.float32),
            pltpu.VMEM((C, D), jnp.float32),
            pltpu.VMEM((C, D), jnp.float32),
            pltpu.SemaphoreType.DMA,
            pltpu.SemaphoreType.DMA,
            pltpu.SemaphoreType.DMA((2 * G,)),
            pltpu.SemaphoreType.DMA((2 * G,)),
            pltpu.SemaphoreType.DMA,
            pltpu.SemaphoreType.DMA,
        ],
    )
    def k(idx_hbm, e1_hbm, e2_hbm, out_hbm,
          x0, x1, a0, a1, b0, b1, p0, p1, q0, q1,
          si0, si1, sg0, sg1, so0, so1):
        wid = lax.axis_index("s") * 2 + lax.axis_index("c")
        base_w = wid * per_w
        rbase_w = base_w // IW

        def issue_idx(ci, xr, si):
            pltpu.async_copy(idx_hbm.at[pl.ds(rbase_w + ci * G, G)], xr, si)

        def wait_idx(xr, si):
            pltpu.make_async_copy(idx_hbm.at[pl.ds(0, G)], xr, si).wait()

        def compute_i(xr, ar, br):
            for j in range(G):
                def grp(g, c):
                    v = xr[j, pl.ds(g * L, L)]
                    ar[j, pl.ds(g * L, L)] = jnp.minimum(v, half - 1)
                    br[j, pl.ds(g * L, L)] = jnp.maximum(v - half, 0)
                    return c
                lax.fori_loop(0, IW // L, grp, 0)

        def issue_gathers(ar, br, pr, qr, sg):
            for j in range(G):
                pltpu.async_copy(e1_hbm.at[ar.at[j]],
                                 pr.at[pl.ds(j * IW, IW)], sg.at[2 * j])
                pltpu.async_copy(e2_hbm.at[br.at[j]],
                                 qr.at[pl.ds(j * IW, IW)], sg.at[2 * j + 1])

        def wait_gathers(ar, br, pr, qr, sg):
            for j in range(G):
                pltpu.make_async_copy(
                    e1_hbm.at[ar.at[j]],
                    pr.at[pl.ds(j * IW, IW)], sg.at[2 * j]).wait()
                pltpu.make_async_copy(
                    e2_hbm.at[br.at[j]],
                    qr.at[pl.ds(j * IW, IW)], sg.at[2 * j + 1]).wait()

        def select(xr, pr, qr):
            for j in range(G):
                def sgrp(g, c):
                    iv = xr[j, pl.ds(g * L, L)]
                    rb = j * IW + g * L
                    for t in range(L):
                        r = rb + t
                        take_e1 = iv[t] < half
                        lo = jnp.where(take_e1, pr[r, pl.ds(0, L)],
                                       qr[r, pl.ds(0, L)])
                        hi = jnp.where(take_e1, pr[r, pl.ds(L, L)],
                                       qr[r, pl.ds(L, L)])
                        pr[r, pl.ds(0, L)] = lo
                        pr[r, pl.ds(L, L)] = hi
                    return c
                lax.fori_loop(0, IW // L, sgrp, 0)

        def issue_out(ci, pr, so):
            pltpu.async_copy(pr, out_hbm.at[pl.ds(base_w + ci * C, C)], so)

        def wait_out(pr, so):
            pltpu.make_async_copy(pr, out_hbm.at[pl.ds(0, C)], so).wait()

        sets = (
            (x0, a0, b0, p0, q0, si0, sg0, so0),
            (x1, a1, b1, p1, q1, si1, sg1, so1),
        )

        issue_idx(0, x0, si0)
        wait_idx(x0, si0)
        compute_i(x0, a0, b0)
        issue_gathers(a0, b0, p0, q0, sg0)
        issue_idx(1, x1, si1)

        def iter_body(ci, A, B):
            xA, aA, bA, pA, qA, siA, sgA, soA = A
            xB, aB, bB, pB, qB, siB, sgB, soB = B

            @pl.when(ci + 1 < n_chunks)
            def _():
                wait_idx(xB, siB)
                compute_i(xB, aB, bB)

                @pl.when(ci >= 1)
                def _():
                    wait_out(pB, soB)
                issue_gathers(aB, bB, pB, qB, sgB)

            wait_gathers(aA, bA, pA, qA, sgA)
            select(xA, pA, qA)
            issue_out(ci, pA, soA)

            @pl.when(ci + 2 < n_chunks)
            def _():
                issue_idx(ci + 2, xA, siA)

        def half_body(h, carry):
            iter_body(2 * h, sets[0], sets[1])
            iter_body(2 * h + 1, sets[1], sets[0])
            return carry

        lax.fori_loop(0, n_chunks // 2, half_body, 0)

        wait_out(p0, so0)
        wait_out(p1, so1)

    return k(idx2d, e1, e2)


def kernel(inputs, e1, e2):
    b, h = inputs.shape
    half = e1.shape[0]
    idx2d = inputs.reshape(b * h // IW, IW).astype(jnp.int32)
    out = _embed_lookup(idx2d, e1, e2, half)
    return out.reshape(b, h, D)


# speedup vs baseline: 1.0012x; 1.0010x over previous
import functools

import jax
import jax.numpy as jnp
from jax import lax
from jax.experimental import pallas as pl
from jax.experimental.pallas import tpu as pltpu
from jax.experimental.pallas import tpu_sc as plsc

D = 32
L = 16
NW = 32
IW = 128
G = 5
C = G * IW


def _embed_lookup(idx2d, e1, e2, half):
    n_rows, _ = idx2d.shape
    n = n_rows * IW
    per_w = n // NW
    n_chunks = per_w // C
    assert n_chunks % 2 == 0

    mesh = plsc.VectorSubcoreMesh(core_axis_name="c", subcore_axis_name="s")

    @functools.partial(
        pl.kernel,
        mesh=mesh,
        compiler_params=pltpu.CompilerParams(use_tc_tiling_on_sc=False),
        out_type=jax.ShapeDtypeStruct((n, D), jnp.float32),
        scratch_types=[
            pltpu.VMEM((G, IW), jnp.int32),
            pltpu.VMEM((G, IW), jnp.int32),
            pltpu.VMEM((G, IW), jnp.int32),
            pltpu.VMEM((G, IW), jnp.int32),
            pltpu.VMEM((G, IW), jnp.int32),
            pltpu.VMEM((G, IW), jnp.int32),
            pltpu.VMEM((C, D), jnp.float32),
            pltpu.VMEM((C, D), jnp.float32),
            pltpu.VMEM((C, D), jnp.float32),
            pltpu.VMEM((C, D), jnp.float32),
            pltpu.SemaphoreType.DMA,
            pltpu.SemaphoreType.DMA,
            pltpu.SemaphoreType.DMA((2 * G,)),
            pltpu.SemaphoreType.DMA((2 * G,)),
            pltpu.SemaphoreType.DMA,
            pltpu.SemaphoreType.DMA,
        ],
    )
    def k(idx_hbm, e1_hbm, e2_hbm, out_hbm,
          x0, x1, a0, a1, b0, b1, p0, p1, q0, q1,
          si0, si1, sg0, sg1, so0, so1):
        wid = lax.axis_index("s") * 2 + lax.axis_index("c")
        base_w = wid * per_w
        rbase_w = base_w // IW

        def issue_idx(ci, xr, si):
            pltpu.async_copy(idx_hbm.at[pl.ds(rbase_w + ci * G, G)], xr, si)

        def wait_idx(xr, si):
            pltpu.make_async_copy(idx_hbm.at[pl.ds(0, G)], xr, si).wait()

        def compute_i(xr, ar, br):
            for j in range(G):
                def grp(g, c):
                    v = xr[j, pl.ds(g * L, L)]
                    ar[j, pl.ds(g * L, L)] = jnp.minimum(v, half - 1)
                    br[j, pl.ds(g * L, L)] = jnp.maximum(v - half, 0)
                    return c
                lax.fori_loop(0, IW // L, grp, 0)

        def issue_gathers(ar, br, pr, qr, sg):
            for j in range(G):
                pltpu.async_copy(e1_hbm.at[ar.at[j]],
                                 pr.at[pl.ds(j * IW, IW)], sg.at[2 * j])
                pltpu.async_copy(e2_hbm.at[br.at[j]],
                                 qr.at[pl.ds(j * IW, IW)], sg.at[2 * j + 1])

        def wait_gathers(ar, br, pr, qr, sg):
            for j in range(G):
                pltpu.make_async_copy(
                    e1_hbm.at[ar.at[j]],
                    pr.at[pl.ds(j * IW, IW)], sg.at[2 * j]).wait()
                pltpu.make_async_copy(
                    e2_hbm.at[br.at[j]],
                    qr.at[pl.ds(j * IW, IW)], sg.at[2 * j + 1]).wait()

        def select(xr, pr, qr):
            for j in range(G):
                def sgrp(g, c):
                    iv = xr[j, pl.ds(g * L, L)]
                    rb = j * IW + g * L
                    for t in range(L):
                        r = rb + t
                        take_e1 = iv[t] < half
                        lo = jnp.where(take_e1, pr[r, pl.ds(0, L)],
                                       qr[r, pl.ds(0, L)])
                        hi = jnp.where(take_e1, pr[r, pl.ds(L, L)],
                                       qr[r, pl.ds(L, L)])
                        pr[r, pl.ds(0, L)] = lo
                        pr[r, pl.ds(L, L)] = hi
                    return c
                lax.fori_loop(0, IW // L, sgrp, 0)

        def issue_out(ci, pr, so):
            pltpu.async_copy(pr, out_hbm.at[pl.ds(base_w + ci * C, C)], so)

        def wait_out(pr, so):
            pltpu.make_async_copy(pr, out_hbm.at[pl.ds(0, C)], so).wait()

        sets = (
            (x0, a0, b0, p0, q0, si0, sg0, so0),
            (x1, a1, b1, p1, q1, si1, sg1, so1),
        )

        issue_idx(0, x0, si0)
        wait_idx(x0, si0)
        compute_i(x0, a0, b0)
        issue_gathers(a0, b0, p0, q0, sg0)
        issue_idx(1, x1, si1)

        def iter_body(ci, A, B):
            xA, aA, bA, pA, qA, siA, sgA, soA = A
            xB, aB, bB, pB, qB, siB, sgB, soB = B

            @pl.when(ci + 1 < n_chunks)
            def _():
                with jax.named_scope("prep"):
                    wait_idx(xB, siB)
                    compute_i(xB, aB, bB)

                @pl.when(ci >= 1)
                def _():
                    with jax.named_scope("wait_out"):
                        wait_out(pB, soB)
                with jax.named_scope("issue_gathers"):
                    issue_gathers(aB, bB, pB, qB, sgB)

            with jax.named_scope("wait_gathers"):
                wait_gathers(aA, bA, pA, qA, sgA)
            with jax.named_scope("select"):
                select(xA, pA, qA)
            issue_out(ci, pA, soA)

            @pl.when(ci + 2 < n_chunks)
            def _():
                issue_idx(ci + 2, xA, siA)

        def half_body(h, carry):
            iter_body(2 * h, sets[0], sets[1])
            iter_body(2 * h + 1, sets[1], sets[0])
            return carry

        lax.fori_loop(0, n_chunks // 2, half_body, 0)

        wait_out(p0, so0)
        wait_out(p1, so1)

    return k(idx2d, e1, e2)


def kernel(inputs, e1, e2):
    b, h = inputs.shape
    half = e1.shape[0]
    idx2d = inputs.reshape(b * h // IW, IW).astype(jnp.int32)
    out = _embed_lookup(idx2d, e1, e2, half)
    return out.reshape(b, h, D)
